# baseline (speedup 1.0000x reference)
import jax
import jax.numpy as jnp
from jax.experimental import pallas as pl
from jax.experimental.pallas import tpu as pltpu

_ROWS = 100000
_DIM = 128
_NCHUNK = 8
_CHUNK = _ROWS // _NCHUNK


def _dma_body(in_hbm, out_hbm, sems):
    for i in range(_NCHUNK):
        pltpu.make_async_copy(
            in_hbm.at[pl.ds(i * _CHUNK, _CHUNK)],
            out_hbm.at[pl.ds(i * _CHUNK, _CHUNK)],
            sems.at[i],
        ).start()
    for i in range(_NCHUNK):
        pltpu.make_async_copy(
            in_hbm.at[pl.ds(i * _CHUNK, _CHUNK)],
            out_hbm.at[pl.ds(i * _CHUNK, _CHUNK)],
            sems.at[i],
        ).wait()


def kernel(weight):
    return pl.pallas_call(
        _dma_body,
        in_specs=[pl.BlockSpec(memory_space=pl.ANY)],
        out_specs=pl.BlockSpec(memory_space=pl.ANY),
        out_shape=jax.ShapeDtypeStruct((_ROWS, _DIM), jnp.float32),
        scratch_shapes=[pltpu.SemaphoreType.DMA((_NCHUNK,))],
    )(weight)


# speedup vs baseline: 42.4228x; 42.4228x over previous
import jax
import jax.numpy as jnp
from jax.experimental import pallas as pl
from jax.experimental.pallas import tpu as pltpu

_ROWS = 100000
_DIM = 128
_CHUNK_ROWS = 4000
_NCHUNKS = _ROWS // _CHUNK_ROWS
_NBUF = 4


def _copy_body(in_hbm, out_hbm, bufs, in_sems, out_sems):
    def copy_in(c, b):
        return pltpu.make_async_copy(
            in_hbm.at[pl.ds(c * _CHUNK_ROWS, _CHUNK_ROWS)], bufs.at[b],
            in_sems.at[b])

    def copy_out(c, b):
        return pltpu.make_async_copy(
            bufs.at[b], out_hbm.at[pl.ds(c * _CHUNK_ROWS, _CHUNK_ROWS)],
            out_sems.at[b])

    for c in range(min(_NBUF, _NCHUNKS)):
        copy_in(c, c).start()
    for c in range(_NCHUNKS):
        b = c % _NBUF
        copy_in(c, b).wait()
        copy_out(c, b).start()
        nxt = c + _NBUF
        if nxt < _NCHUNKS:
            copy_out(c, b).wait()
            copy_in(nxt, b).start()
    for c in range(max(_NCHUNKS - _NBUF, 0), _NCHUNKS):
        copy_out(c, c % _NBUF).wait()


def kernel(weight):
    return pl.pallas_call(
        _copy_body,
        in_specs=[pl.BlockSpec(memory_space=pl.ANY)],
        out_specs=pl.BlockSpec(memory_space=pl.ANY),
        out_shape=jax.ShapeDtypeStruct((_ROWS, _DIM), jnp.float32),
        scratch_shapes=[
            pltpu.VMEM((_NBUF, _CHUNK_ROWS, _DIM), jnp.float32),
            pltpu.SemaphoreType.DMA((_NBUF,)),
            pltpu.SemaphoreType.DMA((_NBUF,)),
        ],
    )(weight)


# speedup vs baseline: 46.5714x; 1.0978x over previous
import jax
import jax.numpy as jnp
from jax.experimental import pallas as pl
from jax.experimental.pallas import tpu as pltpu

_ROWS = 100000
_DIM = 128
_CHUNK_ROWS = 2000
_NCHUNKS = _ROWS // _CHUNK_ROWS
_NBUF = 6
_SHIFT = 2


def _copy_body(in_hbm, out_hbm, bufs, in_sems, out_sems):
    def copy_in(c, b):
        return pltpu.make_async_copy(
            in_hbm.at[pl.ds(c * _CHUNK_ROWS, _CHUNK_ROWS)], bufs.at[b],
            in_sems.at[b])

    def copy_out(c, b):
        return pltpu.make_async_copy(
            bufs.at[b], out_hbm.at[pl.ds(c * _CHUNK_ROWS, _CHUNK_ROWS)],
            out_sems.at[b])

    for c in range(min(_NBUF, _NCHUNKS)):
        copy_in(c, c).start()
    for c in range(_NCHUNKS):
        copy_in(c, c % _NBUF).wait()
        copy_out(c, c % _NBUF).start()
        d = c - _SHIFT
        if d >= 0 and d + _NBUF < _NCHUNKS:
            copy_out(d, d % _NBUF).wait()
            copy_in(d + _NBUF, d % _NBUF).start()
    for c in range(max(_NCHUNKS - _NBUF, 0), _NCHUNKS):
        copy_out(c, c % _NBUF).wait()


def kernel(weight):
    return pl.pallas_call(
        _copy_body,
        in_specs=[pl.BlockSpec(memory_space=pl.ANY)],
        out_specs=pl.BlockSpec(memory_space=pl.ANY),
        out_shape=jax.ShapeDtypeStruct((_ROWS, _DIM), jnp.float32),
        scratch_shapes=[
            pltpu.VMEM((_NBUF, _CHUNK_ROWS, _DIM), jnp.float32),
            pltpu.SemaphoreType.DMA((_NBUF,)),
            pltpu.SemaphoreType.DMA((_NBUF,)),
        ],
    )(weight)


# speedup vs baseline: 48.7865x; 1.0476x over previous
import jax
import jax.numpy as jnp
from jax.experimental import pallas as pl
from jax.experimental.pallas import tpu as pltpu

_ROWS = 100000
_DIM = 128
_CHUNK_ROWS = 5000
_NCHUNKS = _ROWS // _CHUNK_ROWS
_NBUF = 6
_SHIFT = 3


def _copy_body(in_hbm, out_hbm, bufs, in_sems, out_sems):
    def copy_in(c, b):
        return pltpu.make_async_copy(
            in_hbm.at[pl.ds(c * _CHUNK_ROWS, _CHUNK_ROWS)], bufs.at[b],
            in_sems.at[b])

    def copy_out(c, b):
        return pltpu.make_async_copy(
            bufs.at[b], out_hbm.at[pl.ds(c * _CHUNK_ROWS, _CHUNK_ROWS)],
            out_sems.at[b])

    for c in range(min(_NBUF, _NCHUNKS)):
        copy_in(c, c).start()
    for c in range(_NCHUNKS):
        copy_in(c, c % _NBUF).wait()
        copy_out(c, c % _NBUF).start()
        d = c - _SHIFT
        if d >= 0 and d + _NBUF < _NCHUNKS:
            copy_out(d, d % _NBUF).wait()
            copy_in(d + _NBUF, d % _NBUF).start()
    for c in range(max(_NCHUNKS - _NBUF, 0), _NCHUNKS):
        copy_out(c, c % _NBUF).wait()


def kernel(weight):
    return pl.pallas_call(
        _copy_body,
        in_specs=[pl.BlockSpec(memory_space=pl.ANY)],
        out_specs=pl.BlockSpec(memory_space=pl.ANY),
        out_shape=jax.ShapeDtypeStruct((_ROWS, _DIM), jnp.float32),
        scratch_shapes=[
            pltpu.VMEM((_NBUF, _CHUNK_ROWS, _DIM), jnp.float32),
            pltpu.SemaphoreType.DMA((_NBUF,)),
            pltpu.SemaphoreType.DMA((_NBUF,)),
        ],
    )(weight)
